# Initial kernel scaffold; baseline (speedup 1.0000x reference)
import functools

import jax
import jax.numpy as jnp
from jax.experimental import pallas as pl
from jax.experimental.pallas import tpu as pltpu

NUM_EXPERTS = 8
TOP_K = 2
D_MODEL = 1024
D_FF = 2816
LAMBDA_BALANCE = 0.01

F_BLK = 256
T_BLK = 512


def _router_body(x_ref, gate_ref, wfull_ref, loss_ref):
    x = x_ref[...]
    gate = gate_ref[...]
    logits = jax.lax.dot_general(
        x, gate, (((1,), (1,)), ((), ())),
        preferred_element_type=jnp.float32)
    m = jnp.max(logits, axis=1, keepdims=True)
    p = jnp.exp(logits - m)
    rw = p / jnp.sum(p, axis=1, keepdims=True)

    E = rw.shape[1]
    col = jax.lax.broadcasted_iota(jnp.int32, rw.shape, 1)
    m1 = jnp.max(rw, axis=1, keepdims=True)
    a1 = jnp.min(jnp.where(rw == m1, col, E), axis=1, keepdims=True)
    rw2 = jnp.where(col == a1, -jnp.inf, rw)
    m2 = jnp.max(rw2, axis=1, keepdims=True)
    a2 = jnp.min(jnp.where(rw2 == m2, col, E), axis=1, keepdims=True)

    s = m1 + m2
    wfull = jnp.where(col == a1, m1 / s, 0.0) + jnp.where(col == a2, m2 / s, 0.0)
    wfull_ref[...] = wfull

    onehot = (col == a1).astype(jnp.float32) + (col == a2).astype(jnp.float32)
    counts = jnp.sum(onehot, axis=0)
    n_assign = jnp.float32(rw.shape[0] * TOP_K)
    f_i = counts / (n_assign + 1e-06)
    i_i = jnp.mean(rw, axis=0)
    loss_ref[0, 0] = LAMBDA_BALANCE * E * jnp.sum(f_i * i_i)


def _ffn_body(x_ref, wfull_ref, w1_ref, w3_ref, w2_ref, out_ref):
    e = pl.program_id(1)
    f = pl.program_id(2)

    @pl.when(jnp.logical_and(e == 0, f == 0))
    def _():
        out_ref[...] = jnp.zeros_like(out_ref)

    x = x_ref[...]
    w1 = w1_ref[0]
    w3 = w3_ref[0]
    w2 = w2_ref[0]
    a = jax.lax.dot_general(x, w1, (((1,), (1,)), ((), ())),
                            preferred_element_type=jnp.float32)
    b = jax.lax.dot_general(x, w3, (((1,), (1,)), ((), ())),
                            preferred_element_type=jnp.float32)
    h = (a * jax.lax.logistic(a)) * b
    w_e = wfull_ref[...][:, e][:, None]
    h = h * w_e
    out_ref[...] += jax.lax.dot_general(
        h, w2, (((1,), (1,)), ((), ())),
        preferred_element_type=jnp.float32)


def kernel(hidden_states, gate_w, w1, w3, w2):
    B, T, C = hidden_states.shape
    x = hidden_states.reshape(-1, C)
    N = x.shape[0]

    wfull, loss = pl.pallas_call(
        _router_body,
        out_shape=(
            jax.ShapeDtypeStruct((N, NUM_EXPERTS), jnp.float32),
            jax.ShapeDtypeStruct((1, 1), jnp.float32),
        ),
        in_specs=[
            pl.BlockSpec((N, C), lambda: (0, 0)),
            pl.BlockSpec((NUM_EXPERTS, C), lambda: (0, 0)),
        ],
        out_specs=(
            pl.BlockSpec((N, NUM_EXPERTS), lambda: (0, 0)),
            pl.BlockSpec(memory_space=pltpu.SMEM),
        ),
    )(x, gate_w)

    nt = N // T_BLK
    nf = D_FF // F_BLK
    out = pl.pallas_call(
        _ffn_body,
        grid=(nt, NUM_EXPERTS, nf),
        out_shape=jax.ShapeDtypeStruct((N, C), jnp.float32),
        in_specs=[
            pl.BlockSpec((T_BLK, C), lambda t, e, f: (t, 0)),
            pl.BlockSpec((T_BLK, NUM_EXPERTS), lambda t, e, f: (t, 0)),
            pl.BlockSpec((1, F_BLK, C), lambda t, e, f: (e, f, 0)),
            pl.BlockSpec((1, F_BLK, C), lambda t, e, f: (e, f, 0)),
            pl.BlockSpec((1, C, F_BLK), lambda t, e, f: (e, 0, f)),
        ],
        out_specs=pl.BlockSpec((T_BLK, C), lambda t, e, f: (t, 0)),
    )(x, wfull, w1, w3, w2)

    return out.reshape(B, T, C), loss[0, 0]


# speedup vs baseline: 1.1080x; 1.1080x over previous
import functools

import jax
import jax.numpy as jnp
from jax.experimental import pallas as pl
from jax.experimental.pallas import tpu as pltpu

NUM_EXPERTS = 8
TOP_K = 2
D_MODEL = 1024
D_FF = 2816
LAMBDA_BALANCE = 0.01

F_BLK = 256
T_BLK = 512


def _router_body(x_ref, gate_ref, wfull_ref, loss_ref):
    x = x_ref[...]
    gate = gate_ref[...]
    logits = jax.lax.dot_general(
        x, gate, (((1,), (1,)), ((), ())),
        preferred_element_type=jnp.float32)
    m = jnp.max(logits, axis=1, keepdims=True)
    p = jnp.exp(logits - m)
    rw = p / jnp.sum(p, axis=1, keepdims=True)

    E = rw.shape[1]
    col = jax.lax.broadcasted_iota(jnp.int32, rw.shape, 1)
    m1 = jnp.max(rw, axis=1, keepdims=True)
    a1 = jnp.min(jnp.where(rw == m1, col, E), axis=1, keepdims=True)
    rw2 = jnp.where(col == a1, -jnp.inf, rw)
    m2 = jnp.max(rw2, axis=1, keepdims=True)
    a2 = jnp.min(jnp.where(rw2 == m2, col, E), axis=1, keepdims=True)

    s = m1 + m2
    wfull = jnp.where(col == a1, m1 / s, 0.0) + jnp.where(col == a2, m2 / s, 0.0)
    wfull_ref[...] = wfull

    onehot = (col == a1).astype(jnp.float32) + (col == a2).astype(jnp.float32)
    counts = jnp.sum(onehot, axis=0)
    n_assign = jnp.float32(rw.shape[0] * TOP_K)
    f_i = counts / (n_assign + 1e-06)
    i_i = jnp.mean(rw, axis=0)
    loss_ref[0, 0] = LAMBDA_BALANCE * E * jnp.sum(f_i * i_i)


def _ffn_body(x_ref, wfull_ref, w1_ref, w3_ref, w2_ref, out_ref):
    e = pl.program_id(1)
    f = pl.program_id(2)

    @pl.when(jnp.logical_and(e == 0, f == 0))
    def _():
        out_ref[...] = jnp.zeros_like(out_ref)

    x = x_ref[...]
    w1 = w1_ref[0]
    w3 = w3_ref[0]
    w2 = w2_ref[0]
    a = jax.lax.dot_general(x, w1, (((1,), (1,)), ((), ())),
                            preferred_element_type=jnp.float32)
    b = jax.lax.dot_general(x, w3, (((1,), (1,)), ((), ())),
                            preferred_element_type=jnp.float32)
    h = (a * jax.lax.logistic(a)) * b
    wfull = wfull_ref[...]
    ecol = jax.lax.broadcasted_iota(jnp.int32, wfull.shape, 1)
    w_e = jnp.sum(jnp.where(ecol == e, wfull, 0.0), axis=1, keepdims=True)
    h = h * w_e
    out_ref[...] += jax.lax.dot_general(
        h, w2, (((1,), (1,)), ((), ())),
        preferred_element_type=jnp.float32)


def kernel(hidden_states, gate_w, w1, w3, w2):
    B, T, C = hidden_states.shape
    x = hidden_states.reshape(-1, C)
    N = x.shape[0]

    wfull, loss = pl.pallas_call(
        _router_body,
        out_shape=(
            jax.ShapeDtypeStruct((N, NUM_EXPERTS), jnp.float32),
            jax.ShapeDtypeStruct((1, 1), jnp.float32),
        ),
        in_specs=[
            pl.BlockSpec((N, C), lambda: (0, 0)),
            pl.BlockSpec((NUM_EXPERTS, C), lambda: (0, 0)),
        ],
        out_specs=(
            pl.BlockSpec((N, NUM_EXPERTS), lambda: (0, 0)),
            pl.BlockSpec(memory_space=pltpu.SMEM),
        ),
    )(x, gate_w)

    nt = N // T_BLK
    nf = D_FF // F_BLK
    out = pl.pallas_call(
        _ffn_body,
        grid=(nt, NUM_EXPERTS, nf),
        out_shape=jax.ShapeDtypeStruct((N, C), jnp.float32),
        in_specs=[
            pl.BlockSpec((T_BLK, C), lambda t, e, f: (t, 0)),
            pl.BlockSpec((T_BLK, NUM_EXPERTS), lambda t, e, f: (t, 0)),
            pl.BlockSpec((1, F_BLK, C), lambda t, e, f: (e, f, 0)),
            pl.BlockSpec((1, F_BLK, C), lambda t, e, f: (e, f, 0)),
            pl.BlockSpec((1, C, F_BLK), lambda t, e, f: (e, 0, f)),
        ],
        out_specs=pl.BlockSpec((T_BLK, C), lambda t, e, f: (t, 0)),
    )(x, wfull, w1, w3, w2)

    return out.reshape(B, T, C), loss[0, 0]


# speedup vs baseline: 1.1120x; 1.0036x over previous
import functools

import jax
import jax.numpy as jnp
from jax.experimental import pallas as pl
from jax.experimental.pallas import tpu as pltpu

NUM_EXPERTS = 8
TOP_K = 2
D_MODEL = 1024
D_FF = 2816
LAMBDA_BALANCE = 0.01

F_BLK = 256
T_BLK = 512


def _router_body(x_ref, gate_ref, wfull_ref, loss_ref):
    x = x_ref[...]
    gate = gate_ref[...]
    logits = jax.lax.dot_general(
        x, gate, (((1,), (1,)), ((), ())),
        preferred_element_type=jnp.float32)
    m = jnp.max(logits, axis=1, keepdims=True)
    p = jnp.exp(logits - m)
    rw = p / jnp.sum(p, axis=1, keepdims=True)

    E = rw.shape[1]
    col = jax.lax.broadcasted_iota(jnp.int32, rw.shape, 1)
    m1 = jnp.max(rw, axis=1, keepdims=True)
    a1 = jnp.min(jnp.where(rw == m1, col, E), axis=1, keepdims=True)
    rw2 = jnp.where(col == a1, -jnp.inf, rw)
    m2 = jnp.max(rw2, axis=1, keepdims=True)
    a2 = jnp.min(jnp.where(rw2 == m2, col, E), axis=1, keepdims=True)

    s = m1 + m2
    wfull = jnp.where(col == a1, m1 / s, 0.0) + jnp.where(col == a2, m2 / s, 0.0)
    wfull_ref[...] = wfull

    onehot = (col == a1).astype(jnp.float32) + (col == a2).astype(jnp.float32)
    counts = jnp.sum(onehot, axis=0)
    n_assign = jnp.float32(rw.shape[0] * TOP_K)
    f_i = counts / (n_assign + 1e-06)
    i_i = jnp.mean(rw, axis=0)
    loss_ref[0, 0] = LAMBDA_BALANCE * E * jnp.sum(f_i * i_i)


def _ffn_body(x_ref, wfull_ref, w1_ref, w3_ref, w2_ref, out_ref):
    e = pl.program_id(1)
    f = pl.program_id(2)

    @pl.when(jnp.logical_and(e == 0, f == 0))
    def _():
        out_ref[...] = jnp.zeros_like(out_ref)

    x = x_ref[...].astype(jnp.bfloat16)
    w1 = w1_ref[0].astype(jnp.bfloat16)
    w3 = w3_ref[0].astype(jnp.bfloat16)
    w2 = w2_ref[0]
    a = jax.lax.dot_general(x, w1, (((1,), (1,)), ((), ())),
                            preferred_element_type=jnp.float32)
    b = jax.lax.dot_general(x, w3, (((1,), (1,)), ((), ())),
                            preferred_element_type=jnp.float32)
    h = (a * jax.lax.logistic(a)) * b
    wfull = wfull_ref[...]
    ecol = jax.lax.broadcasted_iota(jnp.int32, wfull.shape, 1)
    w_e = jnp.sum(jnp.where(ecol == e, wfull, 0.0), axis=1, keepdims=True)
    h = (h * w_e).astype(jnp.bfloat16)
    out_ref[...] += jax.lax.dot_general(
        h, w2.astype(jnp.bfloat16), (((1,), (1,)), ((), ())),
        preferred_element_type=jnp.float32)


def kernel(hidden_states, gate_w, w1, w3, w2):
    B, T, C = hidden_states.shape
    x = hidden_states.reshape(-1, C)
    N = x.shape[0]

    wfull, loss = pl.pallas_call(
        _router_body,
        out_shape=(
            jax.ShapeDtypeStruct((N, NUM_EXPERTS), jnp.float32),
            jax.ShapeDtypeStruct((1, 1), jnp.float32),
        ),
        in_specs=[
            pl.BlockSpec((N, C), lambda: (0, 0)),
            pl.BlockSpec((NUM_EXPERTS, C), lambda: (0, 0)),
        ],
        out_specs=(
            pl.BlockSpec((N, NUM_EXPERTS), lambda: (0, 0)),
            pl.BlockSpec(memory_space=pltpu.SMEM),
        ),
    )(x, gate_w)

    nt = N // T_BLK
    nf = D_FF // F_BLK
    out = pl.pallas_call(
        _ffn_body,
        grid=(nt, NUM_EXPERTS, nf),
        out_shape=jax.ShapeDtypeStruct((N, C), jnp.float32),
        in_specs=[
            pl.BlockSpec((T_BLK, C), lambda t, e, f: (t, 0)),
            pl.BlockSpec((T_BLK, NUM_EXPERTS), lambda t, e, f: (t, 0)),
            pl.BlockSpec((1, F_BLK, C), lambda t, e, f: (e, f, 0)),
            pl.BlockSpec((1, F_BLK, C), lambda t, e, f: (e, f, 0)),
            pl.BlockSpec((1, C, F_BLK), lambda t, e, f: (e, 0, f)),
        ],
        out_specs=pl.BlockSpec((T_BLK, C), lambda t, e, f: (t, 0)),
    )(x, wfull, w1, w3, w2)

    return out.reshape(B, T, C), loss[0, 0]


# speedup vs baseline: 1.7551x; 1.5783x over previous
import functools

import jax
import jax.numpy as jnp
from jax.experimental import pallas as pl
from jax.experimental.pallas import tpu as pltpu

NUM_EXPERTS = 8
TOP_K = 2
D_MODEL = 1024
D_FF = 2816
LAMBDA_BALANCE = 0.01

F_BLK = 256
T_BLK = 2048


def _router_body(x_ref, gate_ref, wfull_ref, loss_ref):
    x = x_ref[...]
    gate = gate_ref[...]
    logits = jax.lax.dot_general(
        x, gate, (((1,), (1,)), ((), ())),
        preferred_element_type=jnp.float32)
    m = jnp.max(logits, axis=1, keepdims=True)
    p = jnp.exp(logits - m)
    rw = p / jnp.sum(p, axis=1, keepdims=True)

    E = rw.shape[1]
    col = jax.lax.broadcasted_iota(jnp.int32, rw.shape, 1)
    m1 = jnp.max(rw, axis=1, keepdims=True)
    a1 = jnp.min(jnp.where(rw == m1, col, E), axis=1, keepdims=True)
    rw2 = jnp.where(col == a1, -jnp.inf, rw)
    m2 = jnp.max(rw2, axis=1, keepdims=True)
    a2 = jnp.min(jnp.where(rw2 == m2, col, E), axis=1, keepdims=True)

    s = m1 + m2
    wfull = jnp.where(col == a1, m1 / s, 0.0) + jnp.where(col == a2, m2 / s, 0.0)
    wfull_ref[...] = wfull

    onehot = (col == a1).astype(jnp.float32) + (col == a2).astype(jnp.float32)
    counts = jnp.sum(onehot, axis=0)
    n_assign = jnp.float32(rw.shape[0] * TOP_K)
    f_i = counts / (n_assign + 1e-06)
    i_i = jnp.mean(rw, axis=0)
    loss_ref[0, 0] = LAMBDA_BALANCE * E * jnp.sum(f_i * i_i)


def _ffn_body(x_ref, wfull_ref, w1_ref, w3_ref, w2_ref, out_ref):
    e = pl.program_id(1)
    f = pl.program_id(2)

    @pl.when(jnp.logical_and(e == 0, f == 0))
    def _():
        out_ref[...] = jnp.zeros_like(out_ref)

    x = x_ref[...].astype(jnp.bfloat16)
    w1 = w1_ref[0].astype(jnp.bfloat16)
    w3 = w3_ref[0].astype(jnp.bfloat16)
    w2 = w2_ref[0]
    a = jax.lax.dot_general(x, w1, (((1,), (1,)), ((), ())),
                            preferred_element_type=jnp.float32)
    b = jax.lax.dot_general(x, w3, (((1,), (1,)), ((), ())),
                            preferred_element_type=jnp.float32)
    h = (a * jax.lax.logistic(a)) * b
    wfull = wfull_ref[...]
    ecol = jax.lax.broadcasted_iota(jnp.int32, wfull.shape, 1)
    w_e = jnp.sum(jnp.where(ecol == e, wfull, 0.0), axis=1, keepdims=True)
    h = (h * w_e).astype(jnp.bfloat16)
    out_ref[...] += jax.lax.dot_general(
        h, w2.astype(jnp.bfloat16), (((1,), (1,)), ((), ())),
        preferred_element_type=jnp.float32)


def kernel(hidden_states, gate_w, w1, w3, w2):
    B, T, C = hidden_states.shape
    x = hidden_states.reshape(-1, C)
    N = x.shape[0]

    wfull, loss = pl.pallas_call(
        _router_body,
        out_shape=(
            jax.ShapeDtypeStruct((N, NUM_EXPERTS), jnp.float32),
            jax.ShapeDtypeStruct((1, 1), jnp.float32),
        ),
        in_specs=[
            pl.BlockSpec((N, C), lambda: (0, 0)),
            pl.BlockSpec((NUM_EXPERTS, C), lambda: (0, 0)),
        ],
        out_specs=(
            pl.BlockSpec((N, NUM_EXPERTS), lambda: (0, 0)),
            pl.BlockSpec(memory_space=pltpu.SMEM),
        ),
    )(x, gate_w)

    nt = N // T_BLK
    nf = D_FF // F_BLK
    out = pl.pallas_call(
        _ffn_body,
        grid=(nt, NUM_EXPERTS, nf),
        out_shape=jax.ShapeDtypeStruct((N, C), jnp.float32),
        in_specs=[
            pl.BlockSpec((T_BLK, C), lambda t, e, f: (t, 0)),
            pl.BlockSpec((T_BLK, NUM_EXPERTS), lambda t, e, f: (t, 0)),
            pl.BlockSpec((1, F_BLK, C), lambda t, e, f: (e, f, 0)),
            pl.BlockSpec((1, F_BLK, C), lambda t, e, f: (e, f, 0)),
            pl.BlockSpec((1, C, F_BLK), lambda t, e, f: (e, 0, f)),
        ],
        out_specs=pl.BlockSpec((T_BLK, C), lambda t, e, f: (t, 0)),
    )(x, wfull, w1, w3, w2)

    return out.reshape(B, T, C), loss[0, 0]
